# Initial kernel scaffold; baseline (speedup 1.0000x reference)
import jax
import jax.numpy as jnp
from jax.experimental import pallas as pl


def kernel(x, params):
    raise NotImplementedError("write your pallas kernel here")


# speedup vs baseline: 24.2429x; 24.2429x over previous
import functools

import jax
import jax.numpy as jnp
from jax import lax
from jax.experimental import pallas as pl
from jax.experimental.pallas import tpu as pltpu
from jax.experimental.pallas import tpu_sc as plsc

_HID = 32
_K = 9
_EPS = 1e-5
_SIDE = 16
_N = _SIDE ** 3
_B = 4
_NODES = _B * _N
_RB = 512
_NRB = _N // _RB
_RCH = 2048
_KPAD = 16

_PREC = lax.Precision.DEFAULT


def _ffn_body(x_ref, w_ref, b_ref, g_ref, be_ref, o_ref, *, relu):
    y = lax.dot_general(x_ref[...], w_ref[...], (((1,), (1,)), ((), ())),
                        preferred_element_type=jnp.float32, precision=_PREC)
    y = y + b_ref[...]
    y = g_ref[...] * y / jnp.sqrt(1.0 + _EPS) + be_ref[...]
    if relu:
        y = jnp.maximum(y, 0.0)
    o_ref[...] = y


def _ffn(x, w, b, g, be, relu):
    rows, cin = x.shape
    cout = w.shape[0]
    return pl.pallas_call(
        functools.partial(_ffn_body, relu=relu),
        grid=(rows // _RCH,),
        in_specs=[
            pl.BlockSpec((_RCH, cin), lambda i: (i, 0)),
            pl.BlockSpec((cout, cin), lambda i: (0, 0)),
            pl.BlockSpec((1, cout), lambda i: (0, 0)),
            pl.BlockSpec((1, cout), lambda i: (0, 0)),
            pl.BlockSpec((1, cout), lambda i: (0, 0)),
        ],
        out_specs=pl.BlockSpec((_RCH, cout), lambda i: (i, 0)),
        out_shape=jax.ShapeDtypeStruct((rows, cout), jnp.float32),
    )(x, w, b.reshape(1, -1), g.reshape(1, -1), be.reshape(1, -1))


def _knn_body(f_ref, o_ref):
    b = pl.program_id(0)
    r = pl.program_id(1)
    f = f_ref[0]
    x2 = jnp.sum(f * f, axis=1)
    fr = f_ref[0, pl.ds(r * _RB, _RB), :]
    x2r = jnp.sum(fr * fr, axis=1)
    prod = lax.dot_general(fr, f, (((1,), (1,)), ((), ())),
                           preferred_element_type=jnp.float32, precision=_PREC)
    dist = x2r[:, None] + x2[None, :] - 2.0 * prod
    iota = lax.broadcasted_iota(jnp.int32, (_RB, _N), 1)
    rows = []
    for _ in range(_K):
        m = jnp.min(dist, axis=1, keepdims=True)
        eq = dist == m
        am = jnp.min(jnp.where(eq, iota, _N), axis=1)
        rows.append(am + b * _N)
        dist = jnp.where(iota == am[:, None], jnp.inf, dist)
    idx = jnp.stack(rows, axis=0)
    pad = jnp.zeros((_KPAD - _K, _RB), jnp.int32)
    o_ref[0] = jnp.concatenate([idx, pad], axis=0)


def _knn(f):
    return pl.pallas_call(
        _knn_body,
        grid=(_B, _NRB),
        in_specs=[pl.BlockSpec((1, _N, _HID), lambda b, r: (b, 0, 0))],
        out_specs=pl.BlockSpec((1, _KPAD, _RB), lambda b, r: (b, 0, r)),
        out_shape=jax.ShapeDtypeStruct((_B, _KPAD, _N), jnp.int32),
    )(f)


_NW = 32
_RPW = _NODES // _NW
_CH = 128
_NCH = _RPW // _CH
_WPB = _N // _RPW


def _gather_max(f, idx):
    mesh = plsc.VectorSubcoreMesh(core_axis_name="c", subcore_axis_name="s")

    @functools.partial(
        pl.kernel,
        mesh=mesh,
        out_type=jax.ShapeDtypeStruct((_NODES, _HID), jnp.float32),
        scratch_types=[
            pltpu.VMEM((_KPAD, _CH), jnp.int32),
            pltpu.VMEM((_K * _CH, _HID), jnp.float32),
            pltpu.VMEM((_CH, _HID), jnp.float32),
            pltpu.SemaphoreType.DMA,
        ],
        compiler_params=pltpu.CompilerParams(use_tc_tiling_on_sc=False),
    )
    def run(f_hbm, idx_hbm, out_hbm, idx_v, buf_v, out_v, sem):
        wid = lax.axis_index("s") * 2 + lax.axis_index("c")
        b = wid // _WPB
        col0 = (wid % _WPB) * _RPW

        def chunk(c, carry):
            col = col0 + c * _CH
            pltpu.sync_copy(idx_hbm.at[b, :, pl.ds(col, _CH)], idx_v)
            cps = [
                pltpu.async_copy(f_hbm.at[idx_v.at[k]],
                                 buf_v.at[pl.ds(k * _CH, _CH)], sem)
                for k in range(_K)
            ]
            for cp in cps:
                cp.wait()

            def row(i, carry2):
                for half in range(_HID // 16):
                    s = pl.ds(half * 16, 16)
                    v = buf_v[i, s]
                    for k in range(1, _K):
                        v = jnp.maximum(v, buf_v[k * _CH + i, s])
                    out_v[i, s] = v
                return carry2

            lax.fori_loop(0, _CH, row, 0)
            pltpu.sync_copy(out_v,
                            out_hbm.at[pl.ds(wid * _RPW + c * _CH, _CH)])
            return carry

        lax.fori_loop(0, _NCH, chunk, 0)

    return run(f, idx)


def _post_body(f_ref, mx_ref, sc_ref, gw_ref, gb_ref, gg_ref, gbe_ref,
               w2_ref, b2_ref, g2_ref, be2_ref, o_ref):
    f = f_ref[...]
    gc = jnp.concatenate([f, mx_ref[...] - f], axis=1)
    y = lax.dot_general(gc, gw_ref[...], (((1,), (1,)), ((), ())),
                        preferred_element_type=jnp.float32, precision=_PREC)
    y = y + gb_ref[...]
    y = gg_ref[...] * y / jnp.sqrt(1.0 + _EPS) + gbe_ref[...]
    y = jnp.maximum(y, 0.0)
    h = lax.dot_general(y, w2_ref[...], (((1,), (1,)), ((), ())),
                        preferred_element_type=jnp.float32, precision=_PREC)
    h = h + b2_ref[...]
    h = g2_ref[...] * h / jnp.sqrt(1.0 + _EPS) + be2_ref[...]
    o_ref[...] = jnp.maximum(h + sc_ref[...], 0.0)


def _post(f, mx, shortcut, gw, gb, gg, gbe, w2, b2, g2, be2):
    c2 = 2 * _HID
    return pl.pallas_call(
        _post_body,
        grid=(_NODES // _RCH,),
        in_specs=[
            pl.BlockSpec((_RCH, _HID), lambda i: (i, 0)),
            pl.BlockSpec((_RCH, _HID), lambda i: (i, 0)),
            pl.BlockSpec((_RCH, _HID), lambda i: (i, 0)),
            pl.BlockSpec((c2, c2), lambda i: (0, 0)),
            pl.BlockSpec((1, c2), lambda i: (0, 0)),
            pl.BlockSpec((1, c2), lambda i: (0, 0)),
            pl.BlockSpec((1, c2), lambda i: (0, 0)),
            pl.BlockSpec((_HID, c2), lambda i: (0, 0)),
            pl.BlockSpec((1, _HID), lambda i: (0, 0)),
            pl.BlockSpec((1, _HID), lambda i: (0, 0)),
            pl.BlockSpec((1, _HID), lambda i: (0, 0)),
        ],
        out_specs=pl.BlockSpec((_RCH, _HID), lambda i: (i, 0)),
        out_shape=jax.ShapeDtypeStruct((_NODES, _HID), jnp.float32),
    )(f, mx, shortcut, gw, gb.reshape(1, -1), gg.reshape(1, -1),
      gbe.reshape(1, -1), w2, b2.reshape(1, -1), g2.reshape(1, -1),
      be2.reshape(1, -1))


def kernel(x, params):
    p = params
    xp = jnp.pad(x[:, 0], ((0, 0), (1, 1), (1, 1), (1, 1)))
    pats = [xp[:, dz:dz + _SIDE, dy:dy + _SIDE, dx:dx + _SIDE]
            for dz in range(3) for dy in range(3) for dx in range(3)]
    pats = jnp.stack(pats, axis=-1).reshape(_NODES, 27)
    pats = jnp.pad(pats, ((0, 0), (0, 5)))
    w27 = jnp.pad(p['stem_w'].reshape(_HID, 27), ((0, 0), (0, 5)))

    y = _ffn(pats, w27, p['stem_b'], p['stem_g'], p['stem_be'], relu=True)

    for i in range(2):
        f = _ffn(y, p['b%d_fc1_w' % i].reshape(_HID, _HID),
                 p['b%d_fc1_b' % i], p['b%d_fc1_g' % i], p['b%d_fc1_be' % i],
                 relu=False)
        idx = _knn(f.reshape(_B, _N, _HID))
        mx = _gather_max(f, idx)
        y = _post(f, mx, y,
                  p['b%d_g_w' % i].reshape(2 * _HID, 2 * _HID),
                  p['b%d_g_b' % i], p['b%d_g_g' % i], p['b%d_g_be' % i],
                  p['b%d_fc2_w' % i].reshape(_HID, 2 * _HID),
                  p['b%d_fc2_b' % i], p['b%d_fc2_g' % i], p['b%d_fc2_be' % i])

    return y.reshape(_B, _N, _HID).transpose(0, 2, 1).reshape(
        _B, _HID, _SIDE, _SIDE, _SIDE)


# speedup vs baseline: 28.0812x; 1.1583x over previous
import functools

import jax
import jax.numpy as jnp
from jax import lax
from jax.experimental import pallas as pl
from jax.experimental.pallas import tpu as pltpu
from jax.experimental.pallas import tpu_sc as plsc

_HID = 32
_K = 9
_EPS = 1e-5
_SIDE = 16
_N = _SIDE ** 3
_B = 4
_NODES = _B * _N
_RB = 512
_NRB = _N // _RB
_RCH = 2048
_KPAD = 16

_PREC = lax.Precision.DEFAULT


def _ffn_body(x_ref, w_ref, b_ref, g_ref, be_ref, o_ref, *, relu):
    y = lax.dot_general(x_ref[...], w_ref[...], (((1,), (1,)), ((), ())),
                        preferred_element_type=jnp.float32, precision=_PREC)
    y = y + b_ref[...]
    y = g_ref[...] * y / jnp.sqrt(1.0 + _EPS) + be_ref[...]
    if relu:
        y = jnp.maximum(y, 0.0)
    o_ref[...] = y


def _ffn(x, w, b, g, be, relu):
    rows, cin = x.shape
    cout = w.shape[0]
    return pl.pallas_call(
        functools.partial(_ffn_body, relu=relu),
        grid=(rows // _RCH,),
        in_specs=[
            pl.BlockSpec((_RCH, cin), lambda i: (i, 0)),
            pl.BlockSpec((cout, cin), lambda i: (0, 0)),
            pl.BlockSpec((1, cout), lambda i: (0, 0)),
            pl.BlockSpec((1, cout), lambda i: (0, 0)),
            pl.BlockSpec((1, cout), lambda i: (0, 0)),
        ],
        out_specs=pl.BlockSpec((_RCH, cout), lambda i: (i, 0)),
        out_shape=jax.ShapeDtypeStruct((rows, cout), jnp.float32),
    )(x, w, b.reshape(1, -1), g.reshape(1, -1), be.reshape(1, -1))


def _knn_body(f_ref, o_ref):
    b = pl.program_id(0)
    r = pl.program_id(1)
    f = f_ref[0]
    x2 = jnp.sum(f * f, axis=1)
    fr = f_ref[0, pl.ds(r * _RB, _RB), :]
    x2r = jnp.sum(fr * fr, axis=1)
    prod = lax.dot_general(fr, f, (((1,), (1,)), ((), ())),
                           preferred_element_type=jnp.float32, precision=_PREC)
    dist = x2r[:, None] + x2[None, :] - 2.0 * prod
    iota = lax.broadcasted_iota(jnp.int32, (_RB, _N), 1).astype(jnp.float32)
    rows = []
    for _ in range(_K):
        m = jnp.min(dist, axis=1, keepdims=True)
        eq = dist == m
        am = jnp.min(jnp.where(eq, iota, jnp.float32(_N)), axis=1)
        rows.append(am.astype(jnp.int32) + b * _N)
        dist = jnp.where(iota == am[:, None], jnp.inf, dist)
    idx = jnp.stack(rows, axis=0)
    pad = jnp.zeros((_KPAD - _K, _RB), jnp.int32)
    o_ref[0] = jnp.concatenate([idx, pad], axis=0)


def _knn(f):
    return pl.pallas_call(
        _knn_body,
        grid=(_B, _NRB),
        in_specs=[pl.BlockSpec((1, _N, _HID), lambda b, r: (b, 0, 0))],
        out_specs=pl.BlockSpec((1, _KPAD, _RB), lambda b, r: (b, 0, r)),
        out_shape=jax.ShapeDtypeStruct((_B, _KPAD, _N), jnp.int32),
    )(f)


_NW = 32
_RPW = _NODES // _NW
_CH = 128
_NCH = _RPW // _CH
_WPB = _N // _RPW


def _gather_max(f, idx):
    mesh = plsc.VectorSubcoreMesh(core_axis_name="c", subcore_axis_name="s")

    @functools.partial(
        pl.kernel,
        mesh=mesh,
        out_type=jax.ShapeDtypeStruct((_NODES, _HID), jnp.float32),
        scratch_types=[
            pltpu.VMEM((_KPAD, _CH), jnp.int32),
            pltpu.VMEM((_K * _CH, _HID), jnp.float32),
            pltpu.VMEM((_CH, _HID), jnp.float32),
            pltpu.SemaphoreType.DMA,
        ],
        compiler_params=pltpu.CompilerParams(use_tc_tiling_on_sc=False),
    )
    def run(f_hbm, idx_hbm, out_hbm, idx_v, buf_v, out_v, sem):
        wid = lax.axis_index("s") * 2 + lax.axis_index("c")
        b = wid // _WPB
        col0 = (wid % _WPB) * _RPW

        def chunk(c, carry):
            col = col0 + c * _CH
            pltpu.sync_copy(idx_hbm.at[b, :, pl.ds(col, _CH)], idx_v)
            cps = [
                pltpu.async_copy(f_hbm.at[idx_v.at[k]],
                                 buf_v.at[pl.ds(k * _CH, _CH)], sem)
                for k in range(_K)
            ]
            for cp in cps:
                cp.wait()

            def row(i, carry2):
                for half in range(_HID // 16):
                    s = pl.ds(half * 16, 16)
                    v = buf_v[i, s]
                    for k in range(1, _K):
                        v = jnp.maximum(v, buf_v[k * _CH + i, s])
                    out_v[i, s] = v
                return carry2

            lax.fori_loop(0, _CH, row, 0)
            pltpu.sync_copy(out_v,
                            out_hbm.at[pl.ds(wid * _RPW + c * _CH, _CH)])
            return carry

        lax.fori_loop(0, _NCH, chunk, 0)

    return run(f, idx)


def _post_body(f_ref, mx_ref, sc_ref, gw_ref, gb_ref, gg_ref, gbe_ref,
               w2_ref, b2_ref, g2_ref, be2_ref, o_ref, *, transposed):
    f = f_ref[...]
    gc = jnp.concatenate([f, mx_ref[...] - f], axis=1)
    y = lax.dot_general(gc, gw_ref[...], (((1,), (1,)), ((), ())),
                        preferred_element_type=jnp.float32, precision=_PREC)
    y = y + gb_ref[...]
    y = gg_ref[...] * y / jnp.sqrt(1.0 + _EPS) + gbe_ref[...]
    y = jnp.maximum(y, 0.0)
    h = lax.dot_general(y, w2_ref[...], (((1,), (1,)), ((), ())),
                        preferred_element_type=jnp.float32, precision=_PREC)
    h = h + b2_ref[...]
    h = g2_ref[...] * h / jnp.sqrt(1.0 + _EPS) + be2_ref[...]
    y = jnp.maximum(h + sc_ref[...], 0.0)
    if transposed:
        o_ref[0] = y.T
    else:
        o_ref[...] = y


def _post(f, mx, shortcut, gw, gb, gg, gbe, w2, b2, g2, be2, transposed):
    c2 = 2 * _HID
    if transposed:
        out_spec = pl.BlockSpec((1, _HID, _RCH),
                                lambda i: (i // (_N // _RCH), 0, i % (_N // _RCH)))
        out_shape = jax.ShapeDtypeStruct((_B, _HID, _N), jnp.float32)
    else:
        out_spec = pl.BlockSpec((_RCH, _HID), lambda i: (i, 0))
        out_shape = jax.ShapeDtypeStruct((_NODES, _HID), jnp.float32)
    return pl.pallas_call(
        functools.partial(_post_body, transposed=transposed),
        grid=(_NODES // _RCH,),
        in_specs=[
            pl.BlockSpec((_RCH, _HID), lambda i: (i, 0)),
            pl.BlockSpec((_RCH, _HID), lambda i: (i, 0)),
            pl.BlockSpec((_RCH, _HID), lambda i: (i, 0)),
            pl.BlockSpec((c2, c2), lambda i: (0, 0)),
            pl.BlockSpec((1, c2), lambda i: (0, 0)),
            pl.BlockSpec((1, c2), lambda i: (0, 0)),
            pl.BlockSpec((1, c2), lambda i: (0, 0)),
            pl.BlockSpec((_HID, c2), lambda i: (0, 0)),
            pl.BlockSpec((1, _HID), lambda i: (0, 0)),
            pl.BlockSpec((1, _HID), lambda i: (0, 0)),
            pl.BlockSpec((1, _HID), lambda i: (0, 0)),
        ],
        out_specs=out_spec,
        out_shape=out_shape,
    )(f, mx, shortcut, gw, gb.reshape(1, -1), gg.reshape(1, -1),
      gbe.reshape(1, -1), w2, b2.reshape(1, -1), g2.reshape(1, -1),
      be2.reshape(1, -1))


def kernel(x, params):
    p = params
    xp = jnp.pad(x[:, 0], ((0, 0), (1, 1), (1, 1), (1, 1)))
    pats = [xp[:, dz:dz + _SIDE, dy:dy + _SIDE, dx:dx + _SIDE]
            for dz in range(3) for dy in range(3) for dx in range(3)]
    pats = jnp.stack(pats, axis=-1).reshape(_NODES, 27)
    pats = jnp.pad(pats, ((0, 0), (0, 5)))
    w27 = jnp.pad(p['stem_w'].reshape(_HID, 27), ((0, 0), (0, 5)))

    y = _ffn(pats, w27, p['stem_b'], p['stem_g'], p['stem_be'], relu=True)

    for i in range(2):
        f = _ffn(y, p['b%d_fc1_w' % i].reshape(_HID, _HID),
                 p['b%d_fc1_b' % i], p['b%d_fc1_g' % i], p['b%d_fc1_be' % i],
                 relu=False)
        idx = _knn(f.reshape(_B, _N, _HID))
        mx = _gather_max(f, idx)
        y = _post(f, mx, y,
                  p['b%d_g_w' % i].reshape(2 * _HID, 2 * _HID),
                  p['b%d_g_b' % i], p['b%d_g_g' % i], p['b%d_g_be' % i],
                  p['b%d_fc2_w' % i].reshape(_HID, 2 * _HID),
                  p['b%d_fc2_b' % i], p['b%d_fc2_g' % i], p['b%d_fc2_be' % i],
                  transposed=(i == 1))

    return y.reshape(_B, _HID, _SIDE, _SIDE, _SIDE)


# speedup vs baseline: 28.1194x; 1.0014x over previous
import functools

import jax
import jax.numpy as jnp
from jax import lax
from jax.experimental import pallas as pl
from jax.experimental.pallas import tpu as pltpu
from jax.experimental.pallas import tpu_sc as plsc

_HID = 32
_K = 9
_EPS = 1e-5
_SIDE = 16
_N = _SIDE ** 3
_B = 4
_NODES = _B * _N
_RB = 512
_NRB = _N // _RB
_RCH = 2048
_KPAD = 16

_PREC = lax.Precision.DEFAULT


def _ffn_body(x_ref, w_ref, b_ref, g_ref, be_ref, o_ref, *, relu):
    y = lax.dot_general(x_ref[...], w_ref[...], (((1,), (1,)), ((), ())),
                        preferred_element_type=jnp.float32, precision=_PREC)
    y = y + b_ref[...]
    y = g_ref[...] * y / jnp.sqrt(1.0 + _EPS) + be_ref[...]
    if relu:
        y = jnp.maximum(y, 0.0)
    o_ref[...] = y


def _ffn(x, w, b, g, be, relu):
    rows, cin = x.shape
    cout = w.shape[0]
    return pl.pallas_call(
        functools.partial(_ffn_body, relu=relu),
        grid=(rows // _RCH,),
        in_specs=[
            pl.BlockSpec((_RCH, cin), lambda i: (i, 0)),
            pl.BlockSpec((cout, cin), lambda i: (0, 0)),
            pl.BlockSpec((1, cout), lambda i: (0, 0)),
            pl.BlockSpec((1, cout), lambda i: (0, 0)),
            pl.BlockSpec((1, cout), lambda i: (0, 0)),
        ],
        out_specs=pl.BlockSpec((_RCH, cout), lambda i: (i, 0)),
        out_shape=jax.ShapeDtypeStruct((rows, cout), jnp.float32),
    )(x, w, b.reshape(1, -1), g.reshape(1, -1), be.reshape(1, -1))


def _knn_body(y_ref, w_ref, b_ref, g_ref, be_ref, o_ref, f_out_ref, f_ref):
    b = pl.program_id(0)
    r = pl.program_id(1)

    @pl.when(r == 0)
    def _compute_fc1():
        f0 = lax.dot_general(y_ref[0], w_ref[...], (((1,), (1,)), ((), ())),
                             preferred_element_type=jnp.float32,
                             precision=_PREC)
        f0 = f0 + b_ref[...]
        f0 = g_ref[...] * f0 / jnp.sqrt(1.0 + _EPS) + be_ref[...]
        f_ref[...] = f0
        f_out_ref[0] = f0

    f = f_ref[...]
    x2 = jnp.sum(f * f, axis=1)
    fr = f_ref[pl.ds(r * _RB, _RB), :]
    x2r = jnp.sum(fr * fr, axis=1)
    prod = lax.dot_general(fr, f, (((1,), (1,)), ((), ())),
                           preferred_element_type=jnp.float32, precision=_PREC)
    dist = x2r[:, None] + x2[None, :] - 2.0 * prod
    iota = lax.broadcasted_iota(jnp.int32, (_RB, _N), 1).astype(jnp.float32)
    rows = []
    for _ in range(_K):
        m = jnp.min(dist, axis=1, keepdims=True)
        eq = dist == m
        am = jnp.min(jnp.where(eq, iota, jnp.float32(_N)), axis=1)
        rows.append(am.astype(jnp.int32) + b * _N)
        dist = jnp.where(iota == am[:, None], jnp.inf, dist)
    idx = jnp.stack(rows, axis=0)
    pad = jnp.zeros((_KPAD - _K, _RB), jnp.int32)
    o_ref[0] = jnp.concatenate([idx, pad], axis=0)


def _knn(y, w1, b1, g1, be1):
    return pl.pallas_call(
        _knn_body,
        grid=(_B, _NRB),
        in_specs=[
            pl.BlockSpec((1, _N, _HID), lambda b, r: (b, 0, 0)),
            pl.BlockSpec((_HID, _HID), lambda b, r: (0, 0)),
            pl.BlockSpec((1, _HID), lambda b, r: (0, 0)),
            pl.BlockSpec((1, _HID), lambda b, r: (0, 0)),
            pl.BlockSpec((1, _HID), lambda b, r: (0, 0)),
        ],
        out_specs=[
            pl.BlockSpec((1, _KPAD, _RB), lambda b, r: (b, 0, r)),
            pl.BlockSpec((1, _N, _HID), lambda b, r: (b, 0, 0)),
        ],
        out_shape=[
            jax.ShapeDtypeStruct((_B, _KPAD, _N), jnp.int32),
            jax.ShapeDtypeStruct((_B, _N, _HID), jnp.float32),
        ],
        scratch_shapes=[pltpu.VMEM((_N, _HID), jnp.float32)],
    )(y, w1, b1.reshape(1, -1), g1.reshape(1, -1), be1.reshape(1, -1))


_NW = 32
_RPW = _NODES // _NW
_CH = 128
_NCH = _RPW // _CH
_WPB = _N // _RPW


def _gather_max(f, idx):
    mesh = plsc.VectorSubcoreMesh(core_axis_name="c", subcore_axis_name="s")

    @functools.partial(
        pl.kernel,
        mesh=mesh,
        out_type=jax.ShapeDtypeStruct((_NODES, _HID), jnp.float32),
        scratch_types=[
            pltpu.VMEM((_KPAD, _CH), jnp.int32),
            pltpu.VMEM((_K * _CH, _HID), jnp.float32),
            pltpu.VMEM((_CH, _HID), jnp.float32),
            pltpu.SemaphoreType.DMA,
        ],
        compiler_params=pltpu.CompilerParams(use_tc_tiling_on_sc=False),
    )
    def run(f_hbm, idx_hbm, out_hbm, idx_v, buf_v, out_v, sem):
        wid = lax.axis_index("s") * 2 + lax.axis_index("c")
        b = wid // _WPB
        col0 = (wid % _WPB) * _RPW

        def chunk(c, carry):
            col = col0 + c * _CH
            pltpu.sync_copy(idx_hbm.at[b, :, pl.ds(col, _CH)], idx_v)
            cps = [
                pltpu.async_copy(f_hbm.at[idx_v.at[k]],
                                 buf_v.at[pl.ds(k * _CH, _CH)], sem)
                for k in range(_K)
            ]
            for cp in cps:
                cp.wait()

            def row(i, carry2):
                for half in range(_HID // 16):
                    s = pl.ds(half * 16, 16)
                    v = buf_v[i, s]
                    for k in range(1, _K):
                        v = jnp.maximum(v, buf_v[k * _CH + i, s])
                    out_v[i, s] = v
                return carry2

            lax.fori_loop(0, _CH, row, 0)
            pltpu.sync_copy(out_v,
                            out_hbm.at[pl.ds(wid * _RPW + c * _CH, _CH)])
            return carry

        lax.fori_loop(0, _NCH, chunk, 0)

    return run(f, idx)


def _post_body(f_ref, mx_ref, sc_ref, gw_ref, gb_ref, gg_ref, gbe_ref,
               w2_ref, b2_ref, g2_ref, be2_ref, o_ref, *, transposed):
    f = f_ref[...]
    gc = jnp.concatenate([f, mx_ref[...] - f], axis=1)
    y = lax.dot_general(gc, gw_ref[...], (((1,), (1,)), ((), ())),
                        preferred_element_type=jnp.float32, precision=_PREC)
    y = y + gb_ref[...]
    y = gg_ref[...] * y / jnp.sqrt(1.0 + _EPS) + gbe_ref[...]
    y = jnp.maximum(y, 0.0)
    h = lax.dot_general(y, w2_ref[...], (((1,), (1,)), ((), ())),
                        preferred_element_type=jnp.float32, precision=_PREC)
    h = h + b2_ref[...]
    h = g2_ref[...] * h / jnp.sqrt(1.0 + _EPS) + be2_ref[...]
    y = jnp.maximum(h + sc_ref[...], 0.0)
    if transposed:
        o_ref[0] = y.T
    else:
        o_ref[...] = y


def _post(f, mx, shortcut, gw, gb, gg, gbe, w2, b2, g2, be2, transposed):
    c2 = 2 * _HID
    if transposed:
        out_spec = pl.BlockSpec((1, _HID, _RCH),
                                lambda i: (i // (_N // _RCH), 0, i % (_N // _RCH)))
        out_shape = jax.ShapeDtypeStruct((_B, _HID, _N), jnp.float32)
    else:
        out_spec = pl.BlockSpec((_RCH, _HID), lambda i: (i, 0))
        out_shape = jax.ShapeDtypeStruct((_NODES, _HID), jnp.float32)
    return pl.pallas_call(
        functools.partial(_post_body, transposed=transposed),
        grid=(_NODES // _RCH,),
        in_specs=[
            pl.BlockSpec((_RCH, _HID), lambda i: (i, 0)),
            pl.BlockSpec((_RCH, _HID), lambda i: (i, 0)),
            pl.BlockSpec((_RCH, _HID), lambda i: (i, 0)),
            pl.BlockSpec((c2, c2), lambda i: (0, 0)),
            pl.BlockSpec((1, c2), lambda i: (0, 0)),
            pl.BlockSpec((1, c2), lambda i: (0, 0)),
            pl.BlockSpec((1, c2), lambda i: (0, 0)),
            pl.BlockSpec((_HID, c2), lambda i: (0, 0)),
            pl.BlockSpec((1, _HID), lambda i: (0, 0)),
            pl.BlockSpec((1, _HID), lambda i: (0, 0)),
            pl.BlockSpec((1, _HID), lambda i: (0, 0)),
        ],
        out_specs=out_spec,
        out_shape=out_shape,
    )(f, mx, shortcut, gw, gb.reshape(1, -1), gg.reshape(1, -1),
      gbe.reshape(1, -1), w2, b2.reshape(1, -1), g2.reshape(1, -1),
      be2.reshape(1, -1))


def kernel(x, params):
    p = params
    xp = jnp.pad(x[:, 0], ((0, 0), (1, 1), (1, 1), (1, 1)))
    pats = [xp[:, dz:dz + _SIDE, dy:dy + _SIDE, dx:dx + _SIDE]
            for dz in range(3) for dy in range(3) for dx in range(3)]
    pats = jnp.stack(pats, axis=-1).reshape(_NODES, 27)
    pats = jnp.pad(pats, ((0, 0), (0, 5)))
    w27 = jnp.pad(p['stem_w'].reshape(_HID, 27), ((0, 0), (0, 5)))

    y = _ffn(pats, w27, p['stem_b'], p['stem_g'], p['stem_be'], relu=True)

    for i in range(2):
        idx, f = _knn(y.reshape(_B, _N, _HID),
                      p['b%d_fc1_w' % i].reshape(_HID, _HID),
                      p['b%d_fc1_b' % i], p['b%d_fc1_g' % i],
                      p['b%d_fc1_be' % i])
        f = f.reshape(_NODES, _HID)
        mx = _gather_max(f, idx)
        y = _post(f, mx, y,
                  p['b%d_g_w' % i].reshape(2 * _HID, 2 * _HID),
                  p['b%d_g_b' % i], p['b%d_g_g' % i], p['b%d_g_be' % i],
                  p['b%d_fc2_w' % i].reshape(_HID, 2 * _HID),
                  p['b%d_fc2_b' % i], p['b%d_fc2_g' % i], p['b%d_fc2_be' % i],
                  transposed=(i == 1))

    return y.reshape(_B, _HID, _SIDE, _SIDE, _SIDE)


# speedup vs baseline: 28.5917x; 1.0168x over previous
import functools

import jax
import jax.numpy as jnp
from jax import lax
from jax.experimental import pallas as pl
from jax.experimental.pallas import tpu as pltpu
from jax.experimental.pallas import tpu_sc as plsc

_HID = 32
_K = 9
_EPS = 1e-5
_SIDE = 16
_N = _SIDE ** 3
_B = 4
_NODES = _B * _N
_RB = 512
_NRB = _N // _RB
_RCH = 2048
_KPAD = 16

_PREC = lax.Precision.DEFAULT


def _ffn_body(x_ref, w_ref, b_ref, g_ref, be_ref, o_ref, *, relu):
    y = lax.dot_general(x_ref[...], w_ref[...], (((1,), (1,)), ((), ())),
                        preferred_element_type=jnp.float32, precision=_PREC)
    y = y + b_ref[...]
    y = g_ref[...] * y / jnp.sqrt(1.0 + _EPS) + be_ref[...]
    if relu:
        y = jnp.maximum(y, 0.0)
    o_ref[...] = y


def _ffn(x, w, b, g, be, relu):
    rows, cin = x.shape
    cout = w.shape[0]
    return pl.pallas_call(
        functools.partial(_ffn_body, relu=relu),
        grid=(rows // _RCH,),
        in_specs=[
            pl.BlockSpec((_RCH, cin), lambda i: (i, 0)),
            pl.BlockSpec((cout, cin), lambda i: (0, 0)),
            pl.BlockSpec((1, cout), lambda i: (0, 0)),
            pl.BlockSpec((1, cout), lambda i: (0, 0)),
            pl.BlockSpec((1, cout), lambda i: (0, 0)),
        ],
        out_specs=pl.BlockSpec((_RCH, cout), lambda i: (i, 0)),
        out_shape=jax.ShapeDtypeStruct((rows, cout), jnp.float32),
    )(x, w, b.reshape(1, -1), g.reshape(1, -1), be.reshape(1, -1))


def _knn_body(y_ref, w_ref, b_ref, g_ref, be_ref, o_ref, f_out_ref, f_ref):
    b = pl.program_id(0)

    f0 = lax.dot_general(y_ref[0], w_ref[...], (((1,), (1,)), ((), ())),
                         preferred_element_type=jnp.float32,
                         precision=_PREC)
    f0 = f0 + b_ref[...]
    f0 = g_ref[...] * f0 / jnp.sqrt(1.0 + _EPS) + be_ref[...]
    f_ref[...] = f0
    f_out_ref[0] = f0

    f = f_ref[...]
    x2 = jnp.sum(f * f, axis=1)
    iota = lax.broadcasted_iota(jnp.int32, (_RB, _N), 1).astype(jnp.float32)

    def rowblock(r, carry):
        fr = f_ref[pl.ds(r * _RB, _RB), :]
        x2r = jnp.sum(fr * fr, axis=1)
        prod = lax.dot_general(fr, f, (((1,), (1,)), ((), ())),
                               preferred_element_type=jnp.float32,
                               precision=_PREC)
        dist = x2r[:, None] + x2[None, :] - 2.0 * prod
        rows = []
        for _ in range(_K):
            m = jnp.min(dist, axis=1, keepdims=True)
            eq = dist == m
            am = jnp.min(jnp.where(eq, iota, jnp.float32(_N)), axis=1)
            rows.append(am.astype(jnp.int32) + b * _N)
            dist = jnp.where(iota == am[:, None], jnp.inf, dist)
        idx = jnp.stack(rows, axis=0)
        pad = jnp.zeros((_KPAD - _K, _RB), jnp.int32)
        o_ref[0, :, pl.ds(r * _RB, _RB)] = jnp.concatenate([idx, pad], axis=0)
        return carry

    lax.fori_loop(0, _NRB, rowblock, 0)


def _knn(y, w1, b1, g1, be1):
    return pl.pallas_call(
        _knn_body,
        grid=(_B,),
        in_specs=[
            pl.BlockSpec((1, _N, _HID), lambda b: (b, 0, 0)),
            pl.BlockSpec((_HID, _HID), lambda b: (0, 0)),
            pl.BlockSpec((1, _HID), lambda b: (0, 0)),
            pl.BlockSpec((1, _HID), lambda b: (0, 0)),
            pl.BlockSpec((1, _HID), lambda b: (0, 0)),
        ],
        out_specs=[
            pl.BlockSpec((1, _KPAD, _N), lambda b: (b, 0, 0)),
            pl.BlockSpec((1, _N, _HID), lambda b: (b, 0, 0)),
        ],
        out_shape=[
            jax.ShapeDtypeStruct((_B, _KPAD, _N), jnp.int32),
            jax.ShapeDtypeStruct((_B, _N, _HID), jnp.float32),
        ],
        scratch_shapes=[pltpu.VMEM((_N, _HID), jnp.float32)],
    )(y, w1, b1.reshape(1, -1), g1.reshape(1, -1), be1.reshape(1, -1))


_NW = 32
_RPW = _NODES // _NW
_CH = 128
_NCH = _RPW // _CH
_WPB = _N // _RPW


def _gather_max(f, idx):
    mesh = plsc.VectorSubcoreMesh(core_axis_name="c", subcore_axis_name="s")

    @functools.partial(
        pl.kernel,
        mesh=mesh,
        out_type=jax.ShapeDtypeStruct((_NODES, _HID), jnp.float32),
        scratch_types=[
            pltpu.VMEM((_KPAD, _CH), jnp.int32),
            pltpu.VMEM((_K * _CH, _HID), jnp.float32),
            pltpu.VMEM((_CH, _HID), jnp.float32),
            pltpu.SemaphoreType.DMA,
        ],
        compiler_params=pltpu.CompilerParams(use_tc_tiling_on_sc=False),
    )
    def run(f_hbm, idx_hbm, out_hbm, idx_v, buf_v, out_v, sem):
        wid = lax.axis_index("s") * 2 + lax.axis_index("c")
        b = wid // _WPB
        col0 = (wid % _WPB) * _RPW

        def chunk(c, carry):
            col = col0 + c * _CH
            pltpu.sync_copy(idx_hbm.at[b, :, pl.ds(col, _CH)], idx_v)
            cps = [
                pltpu.async_copy(f_hbm.at[idx_v.at[k]],
                                 buf_v.at[pl.ds(k * _CH, _CH)], sem)
                for k in range(_K)
            ]
            for cp in cps:
                cp.wait()

            def row(i, carry2):
                for half in range(_HID // 16):
                    s = pl.ds(half * 16, 16)
                    v = buf_v[i, s]
                    for k in range(1, _K):
                        v = jnp.maximum(v, buf_v[k * _CH + i, s])
                    out_v[i, s] = v
                return carry2

            lax.fori_loop(0, _CH, row, 0)
            pltpu.sync_copy(out_v,
                            out_hbm.at[pl.ds(wid * _RPW + c * _CH, _CH)])
            return carry

        lax.fori_loop(0, _NCH, chunk, 0)

    return run(f, idx)


def _post_body(f_ref, mx_ref, sc_ref, gw_ref, gb_ref, gg_ref, gbe_ref,
               w2_ref, b2_ref, g2_ref, be2_ref, o_ref, *, transposed):
    f = f_ref[...]
    gc = jnp.concatenate([f, mx_ref[...] - f], axis=1)
    y = lax.dot_general(gc, gw_ref[...], (((1,), (1,)), ((), ())),
                        preferred_element_type=jnp.float32, precision=_PREC)
    y = y + gb_ref[...]
    y = gg_ref[...] * y / jnp.sqrt(1.0 + _EPS) + gbe_ref[...]
    y = jnp.maximum(y, 0.0)
    h = lax.dot_general(y, w2_ref[...], (((1,), (1,)), ((), ())),
                        preferred_element_type=jnp.float32, precision=_PREC)
    h = h + b2_ref[...]
    h = g2_ref[...] * h / jnp.sqrt(1.0 + _EPS) + be2_ref[...]
    y = jnp.maximum(h + sc_ref[...], 0.0)
    if transposed:
        o_ref[0] = y.T
    else:
        o_ref[...] = y


def _post(f, mx, shortcut, gw, gb, gg, gbe, w2, b2, g2, be2, transposed):
    c2 = 2 * _HID
    if transposed:
        out_spec = pl.BlockSpec((1, _HID, _RCH),
                                lambda i: (i // (_N // _RCH), 0, i % (_N // _RCH)))
        out_shape = jax.ShapeDtypeStruct((_B, _HID, _N), jnp.float32)
    else:
        out_spec = pl.BlockSpec((_RCH, _HID), lambda i: (i, 0))
        out_shape = jax.ShapeDtypeStruct((_NODES, _HID), jnp.float32)
    return pl.pallas_call(
        functools.partial(_post_body, transposed=transposed),
        grid=(_NODES // _RCH,),
        in_specs=[
            pl.BlockSpec((_RCH, _HID), lambda i: (i, 0)),
            pl.BlockSpec((_RCH, _HID), lambda i: (i, 0)),
            pl.BlockSpec((_RCH, _HID), lambda i: (i, 0)),
            pl.BlockSpec((c2, c2), lambda i: (0, 0)),
            pl.BlockSpec((1, c2), lambda i: (0, 0)),
            pl.BlockSpec((1, c2), lambda i: (0, 0)),
            pl.BlockSpec((1, c2), lambda i: (0, 0)),
            pl.BlockSpec((_HID, c2), lambda i: (0, 0)),
            pl.BlockSpec((1, _HID), lambda i: (0, 0)),
            pl.BlockSpec((1, _HID), lambda i: (0, 0)),
            pl.BlockSpec((1, _HID), lambda i: (0, 0)),
        ],
        out_specs=out_spec,
        out_shape=out_shape,
    )(f, mx, shortcut, gw, gb.reshape(1, -1), gg.reshape(1, -1),
      gbe.reshape(1, -1), w2, b2.reshape(1, -1), g2.reshape(1, -1),
      be2.reshape(1, -1))


def kernel(x, params):
    p = params
    xp = jnp.pad(x[:, 0], ((0, 0), (1, 1), (1, 1), (1, 1)))
    pats = [xp[:, dz:dz + _SIDE, dy:dy + _SIDE, dx:dx + _SIDE]
            for dz in range(3) for dy in range(3) for dx in range(3)]
    pats = jnp.stack(pats, axis=-1).reshape(_NODES, 27)
    pats = jnp.pad(pats, ((0, 0), (0, 5)))
    w27 = jnp.pad(p['stem_w'].reshape(_HID, 27), ((0, 0), (0, 5)))

    y = _ffn(pats, w27, p['stem_b'], p['stem_g'], p['stem_be'], relu=True)

    for i in range(2):
        idx, f = _knn(y.reshape(_B, _N, _HID),
                      p['b%d_fc1_w' % i].reshape(_HID, _HID),
                      p['b%d_fc1_b' % i], p['b%d_fc1_g' % i],
                      p['b%d_fc1_be' % i])
        f = f.reshape(_NODES, _HID)
        mx = _gather_max(f, idx)
        y = _post(f, mx, y,
                  p['b%d_g_w' % i].reshape(2 * _HID, 2 * _HID),
                  p['b%d_g_b' % i], p['b%d_g_g' % i], p['b%d_g_be' % i],
                  p['b%d_fc2_w' % i].reshape(_HID, 2 * _HID),
                  p['b%d_fc2_b' % i], p['b%d_fc2_g' % i], p['b%d_fc2_be' % i],
                  transposed=(i == 1))

    return y.reshape(_B, _HID, _SIDE, _SIDE, _SIDE)
